# speedup vs baseline: 1.0076x; 1.0076x over previous
import functools

import jax
import jax.numpy as jnp
from jax import lax
from jax.experimental import pallas as pl
from jax.experimental.pallas import tpu as pltpu
from jax.experimental.pallas import tpu_sc as plsc

D_MODEL = 1024
SEQ = 4096
NUM_CORES = 2
NUM_SUBCORES = 16
NUM_WORKERS = NUM_CORES * NUM_SUBCORES
ROWS_PER_WORKER = SEQ // NUM_WORKERS
CHUNK = 64


@functools.partial(
    pl.kernel,
    out_type=jax.ShapeDtypeStruct((SEQ, D_MODEL), jnp.float32),
    mesh=plsc.VectorSubcoreMesh(core_axis_name="c", subcore_axis_name="s"),
    scratch_types=[
        pltpu.VMEM((ROWS_PER_WORKER,), jnp.int32),
        pltpu.VMEM((CHUNK, D_MODEL), jnp.float32),
        pltpu.SemaphoreType.DMA,
    ],
)
def _sc_gather(pe_hbm, idx_hbm, out_hbm, idx_v, rows_v, sem):
    wid = lax.axis_index("s") * NUM_CORES + lax.axis_index("c")
    base = wid * ROWS_PER_WORKER
    pltpu.sync_copy(idx_hbm.at[pl.ds(base, ROWS_PER_WORKER)], idx_v)
    for c in range(ROWS_PER_WORKER // CHUNK):
        pltpu.async_copy(
            pe_hbm.at[idx_v.at[pl.ds(c * CHUNK, CHUNK)]], rows_v, sem
        ).wait()
        pltpu.sync_copy(rows_v, out_hbm.at[pl.ds(base + c * CHUNK, CHUNK)])


def _tc_add_body(x_ref, g_ref, o_ref):
    o_ref[...] = x_ref[...] + g_ref[...][None]


def _tc_add(x, g):
    b, s, d = x.shape
    bs = 512
    return pl.pallas_call(
        _tc_add_body,
        grid=(s // bs,),
        in_specs=[
            pl.BlockSpec((b, bs, d), lambda i: (0, i, 0)),
            pl.BlockSpec((bs, d), lambda i: (i, 0)),
        ],
        out_specs=pl.BlockSpec((b, bs, d), lambda i: (0, i, 0)),
        out_shape=jax.ShapeDtypeStruct((b, s, d), x.dtype),
    )(x, g)


def kernel(x, time_indices, pe):
    idx = time_indices.astype(jnp.int32)
    pe2d = pe.reshape(pe.shape[0], pe.shape[-1])
    gathered = _sc_gather(pe2d, idx)
    return _tc_add(x, gathered)


# speedup vs baseline: 1.3371x; 1.3270x over previous
import functools

import jax
import jax.numpy as jnp
from jax import lax
from jax.experimental import pallas as pl
from jax.experimental.pallas import tpu as pltpu
from jax.experimental.pallas import tpu_sc as plsc

D_MODEL = 1024
SUB = 8
LANES = 128
SEQ = 4096
NUM_CORES = 2
NUM_SUBCORES = 16
NUM_WORKERS = NUM_CORES * NUM_SUBCORES
ROWS_PER_WORKER = SEQ // NUM_WORKERS
CHUNK = 64


@functools.partial(
    pl.kernel,
    out_type=jax.ShapeDtypeStruct((SEQ, SUB, LANES), jnp.float32),
    mesh=plsc.VectorSubcoreMesh(core_axis_name="c", subcore_axis_name="s"),
    scratch_types=[
        pltpu.VMEM((ROWS_PER_WORKER,), jnp.int32),
        pltpu.VMEM((CHUNK, SUB, LANES), jnp.float32),
        pltpu.SemaphoreType.DMA,
    ],
)
def _sc_gather(pe_hbm, idx_hbm, out_hbm, idx_v, rows_v, sem):
    wid = lax.axis_index("s") * NUM_CORES + lax.axis_index("c")
    base = wid * ROWS_PER_WORKER
    pltpu.sync_copy(idx_hbm.at[pl.ds(base, ROWS_PER_WORKER)], idx_v)
    for c in range(ROWS_PER_WORKER // CHUNK):
        pltpu.async_copy(
            pe_hbm.at[idx_v.at[pl.ds(c * CHUNK, CHUNK)]], rows_v, sem
        ).wait()
        pltpu.sync_copy(rows_v, out_hbm.at[pl.ds(base + c * CHUNK, CHUNK)])


def _tc_add_body(x_ref, g_ref, o_ref):
    for j in range(SUB):
        sl = slice(j * LANES, (j + 1) * LANES)
        o_ref[:, :, sl] = x_ref[:, :, sl] + g_ref[:, j, :][None]


def _tc_add(x, g):
    b, s, d = x.shape
    bs = 512
    return pl.pallas_call(
        _tc_add_body,
        grid=(s // bs,),
        in_specs=[
            pl.BlockSpec((b, bs, d), lambda i: (0, i, 0)),
            pl.BlockSpec((bs, SUB, LANES), lambda i: (i, 0, 0)),
        ],
        out_specs=pl.BlockSpec((b, bs, d), lambda i: (0, i, 0)),
        out_shape=jax.ShapeDtypeStruct((b, s, d), x.dtype),
    )(x, g)


def kernel(x, time_indices, pe):
    idx = time_indices.astype(jnp.int32)
    pe3 = pe.reshape(pe.shape[0], SUB, LANES)
    gathered = _sc_gather(pe3, idx)
    return _tc_add(x, gathered)
